# baseline (speedup 1.0000x reference)
import functools

import jax
import jax.numpy as jnp
from jax import lax
from jax.experimental import pallas as pl
from jax.experimental.pallas import tpu as pltpu

NUM_CODES = 8192
DIM = 64
ROW_BLK = 256
COL_BLK = 1024


def _argmin_body(v_ref, cb_ref, idx_ref, vq_ref):
    pid = pl.program_id(0)
    v = v_ref[...]
    vsq = jnp.sum(v * v, axis=1, keepdims=True)
    run_min = jnp.full((ROW_BLK,), jnp.inf, dtype=jnp.float32)
    run_idx = jnp.zeros((ROW_BLK,), dtype=jnp.int32)
    for k in range(NUM_CODES // COL_BLK):
        cb = cb_ref[pl.ds(k * COL_BLK, COL_BLK), :]
        cbsq = jnp.sum(cb * cb, axis=1)
        mm = lax.dot_general(v, cb, (((1,), (1,)), ((), ())),
                             preferred_element_type=jnp.float32)
        d = (vsq + cbsq[None, :]) - 2.0 * mm
        m = jnp.min(d, axis=1)
        iota = lax.broadcasted_iota(jnp.int32, (ROW_BLK, COL_BLK), 1)
        lidx = jnp.min(jnp.where(d == m[:, None], iota + k * COL_BLK,
                                 jnp.int32(2**30)), axis=1)
        upd = m < run_min
        run_idx = jnp.where(upd, lidx, run_idx)
        run_min = jnp.where(upd, m, run_min)
    idx_ref[0, 0, :] = run_idx

    @pl.when(pid == 0)
    def _init():
        vq_ref[...] = jnp.zeros((1, 1), jnp.float32)

    vq_ref[...] += jnp.sum(run_min).reshape(1, 1)


def _nearest(vectors, codebook):
    n = vectors.shape[0]
    nblk = n // ROW_BLK
    idx3, vq_sum = pl.pallas_call(
        _argmin_body,
        grid=(nblk,),
        in_specs=[
            pl.BlockSpec((ROW_BLK, DIM), lambda i: (i, 0)),
            pl.BlockSpec((NUM_CODES, DIM), lambda i: (0, 0)),
        ],
        out_specs=[
            pl.BlockSpec((1, 1, ROW_BLK), lambda i: (i, 0, 0)),
            pl.BlockSpec((1, 1), lambda i: (0, 0)),
        ],
        out_shape=[
            jax.ShapeDtypeStruct((nblk, 1, ROW_BLK), jnp.int32),
            jax.ShapeDtypeStruct((1, 1), jnp.float32),
        ],
    )(vectors, codebook)
    return idx3.reshape(n), vq_sum.reshape(())


def _stats_body(h_ref, perp_ref, act_ref):
    counts = jnp.sum(h_ref[...], axis=0, keepdims=True)
    total = jnp.sum(counts)
    probs = counts / total
    ent = jnp.sum(probs * jnp.log(probs + 1e-10))
    perp_ref[...] = jnp.exp(-ent).reshape(1, 1)
    act_ref[...] = jnp.sum((counts > 0).astype(jnp.float32)).reshape(1, 1)


def _stats(hist):
    perp, act = pl.pallas_call(
        _stats_body,
        out_shape=[
            jax.ShapeDtypeStruct((1, 1), jnp.float32),
            jax.ShapeDtypeStruct((1, 1), jnp.float32),
        ],
    )(hist)
    return perp.reshape(()), act.reshape(())


def kernel(z, codebook):
    B, C, H, W = z.shape
    n = B * H * W
    vectors = jnp.transpose(z, (0, 2, 3, 1)).reshape(n, C)
    indices, vq_sum = _nearest(vectors, codebook)

    qrows = jnp.take(codebook, indices, axis=0)
    hist = jnp.bincount(indices, length=NUM_CODES).astype(jnp.float32)
    hist = hist.reshape(1, NUM_CODES)

    quantized = jnp.transpose(qrows.reshape(B, H, W, C), (0, 3, 1, 2))
    vq_loss = vq_sum / jnp.float32(n * C)
    perplexity, active = _stats(hist)
    return (quantized, indices, vq_loss, perplexity, active)


# speedup vs baseline: 1.1570x; 1.1570x over previous
import functools

import jax
import jax.numpy as jnp
from jax import lax
from jax.experimental import pallas as pl
from jax.experimental.pallas import tpu as pltpu
from jax.experimental.pallas import tpu_sc as plsc

NUM_CODES = 8192
DIM = 64
ROW_BLK = 256
COL_BLK = 1024

_SC_INFO = plsc.get_sparse_core_info()
_NC = _SC_INFO.num_cores
_NS = _SC_INFO.num_subcores
_L = _SC_INFO.num_lanes
_NW = _NC * _NS


def _argmin_body(v_ref, cb_ref, idx_ref, vq_ref):
    pid = pl.program_id(0)
    v = v_ref[...]
    vsq = jnp.sum(v * v, axis=1, keepdims=True)
    run_min = jnp.full((ROW_BLK,), jnp.inf, dtype=jnp.float32)
    run_idx = jnp.zeros((ROW_BLK,), dtype=jnp.int32)
    for k in range(NUM_CODES // COL_BLK):
        cb = cb_ref[pl.ds(k * COL_BLK, COL_BLK), :]
        cbsq = jnp.sum(cb * cb, axis=1)
        mm = lax.dot_general(v, cb, (((1,), (1,)), ((), ())),
                             preferred_element_type=jnp.float32)
        d = (vsq + cbsq[None, :]) - 2.0 * mm
        m = jnp.min(d, axis=1)
        iota = lax.broadcasted_iota(jnp.int32, (ROW_BLK, COL_BLK), 1)
        lidx = jnp.min(jnp.where(d == m[:, None], iota + k * COL_BLK,
                                 jnp.int32(2**30)), axis=1)
        upd = m < run_min
        run_idx = jnp.where(upd, lidx, run_idx)
        run_min = jnp.where(upd, m, run_min)
    idx_ref[0, 0, :] = run_idx

    @pl.when(pid == 0)
    def _init():
        vq_ref[...] = jnp.zeros((1, 1), jnp.float32)

    vq_ref[...] += jnp.sum(run_min).reshape(1, 1)


def _nearest(vectors, codebook):
    n = vectors.shape[0]
    nblk = n // ROW_BLK
    idx3, vq_sum = pl.pallas_call(
        _argmin_body,
        grid=(nblk,),
        in_specs=[
            pl.BlockSpec((ROW_BLK, DIM), lambda i: (i, 0)),
            pl.BlockSpec((NUM_CODES, DIM), lambda i: (0, 0)),
        ],
        out_specs=[
            pl.BlockSpec((1, 1, ROW_BLK), lambda i: (i, 0, 0)),
            pl.BlockSpec((1, 1), lambda i: (0, 0)),
        ],
        out_shape=[
            jax.ShapeDtypeStruct((nblk, 1, ROW_BLK), jnp.int32),
            jax.ShapeDtypeStruct((1, 1), jnp.float32),
        ],
    )(vectors, codebook)
    return idx3.reshape(n), vq_sum.reshape(())


_GCHUNK = 128
_PAD_D = 128


def _sc_gather_hist(n):
    per_w = n // _NW
    mesh = plsc.VectorSubcoreMesh(core_axis_name="c", subcore_axis_name="s")

    @functools.partial(
        pl.kernel,
        mesh=mesh,
        out_type=[
            jax.ShapeDtypeStruct((n, _PAD_D), jnp.float32),
            jax.ShapeDtypeStruct((_NW, NUM_CODES), jnp.float32),
        ],
        scratch_types=[
            pltpu.VMEM((per_w,), jnp.int32),
            pltpu.VMEM((per_w, _PAD_D), jnp.float32),
            pltpu.VMEM((NUM_CODES,), jnp.float32),
            pltpu.SemaphoreType.DMA,
        ],
        compiler_params=pltpu.CompilerParams(needs_layout_passes=False),
    )
    def body(idx_hbm, cb_hbm, rows_out, hist_out, idx_v, rows_v, hist_v, sem):
        wid = lax.axis_index("s") * _NC + lax.axis_index("c")
        base = wid * per_w
        pltpu.sync_copy(idx_hbm.at[pl.ds(base, per_w)], idx_v)
        copies = [
            pltpu.async_copy(
                cb_hbm.at[idx_v.at[pl.ds(k * _GCHUNK, _GCHUNK)]],
                rows_v.at[pl.ds(k * _GCHUNK, _GCHUNK)],
                sem,
            )
            for k in range(per_w // _GCHUNK)
        ]
        for c in copies:
            c.wait()
        pltpu.sync_copy(rows_v, rows_out.at[pl.ds(base, per_w)])

        def _zero(i, carry):
            hist_v[pl.ds(i * _L, _L)] = jnp.zeros((_L,), jnp.float32)
            return carry

        lax.fori_loop(0, NUM_CODES // _L, _zero, 0)
        ones = jnp.ones((_L,), jnp.float32)
        full = jnp.ones((_L,), jnp.bool_)

        def _acc(i, carry):
            iv = idx_v[pl.ds(i * _L, _L)]
            plsc.addupdate_scatter(hist_v, [iv], ones, mask=full)
            return carry

        lax.fori_loop(0, per_w // _L, _acc, 0)
        pltpu.sync_copy(hist_v, hist_out.at[wid])

    return body


def _stats_body(h_ref, perp_ref, act_ref):
    counts = jnp.sum(h_ref[...], axis=0, keepdims=True)
    total = jnp.sum(counts)
    probs = counts / total
    ent = jnp.sum(probs * jnp.log(probs + 1e-10))
    perp_ref[...] = jnp.exp(-ent).reshape(1, 1)
    act_ref[...] = jnp.sum((counts > 0).astype(jnp.float32)).reshape(1, 1)


def _stats(hist):
    perp, act = pl.pallas_call(
        _stats_body,
        out_shape=[
            jax.ShapeDtypeStruct((1, 1), jnp.float32),
            jax.ShapeDtypeStruct((1, 1), jnp.float32),
        ],
    )(hist)
    return perp.reshape(()), act.reshape(())


def kernel(z, codebook):
    B, C, H, W = z.shape
    n = B * H * W
    vectors = jnp.transpose(z, (0, 2, 3, 1)).reshape(n, C)
    indices, vq_sum = _nearest(vectors, codebook)

    cb_pad = jnp.pad(codebook, ((0, 0), (0, _PAD_D - C)))
    qrows, hist = _sc_gather_hist(n)(indices, cb_pad)

    quantized = jnp.transpose(qrows[:, :C].reshape(B, H, W, C), (0, 3, 1, 2))
    vq_loss = vq_sum / jnp.float32(n * C)
    perplexity, active = _stats(hist)
    return (quantized, indices, vq_loss, perplexity, active)


# speedup vs baseline: 2.0440x; 1.7667x over previous
import functools

import jax
import jax.numpy as jnp
from jax import lax
from jax.experimental import pallas as pl
from jax.experimental.pallas import tpu as pltpu
from jax.experimental.pallas import tpu_sc as plsc

NUM_CODES = 8192
DIM = 64
ROW_BLK = 256
COL_BLK = 1024

_SC_INFO = plsc.get_sparse_core_info()
_NC = _SC_INFO.num_cores
_NS = _SC_INFO.num_subcores
_L = _SC_INFO.num_lanes
_NW = _NC * _NS


def _argmin_body(v_ref, cbt_ref, idx_ref, vq_ref, cbsq_ref):
    pid = pl.program_id(0)

    @pl.when(pid == 0)
    def _prep():
        cbt = cbt_ref[...]
        cbsq_ref[...] = jnp.sum(cbt * cbt, axis=0, keepdims=True)

    v = v_ref[...]
    vsq = jnp.sum(v * v, axis=1, keepdims=True)
    v2 = v * (-2.0)
    runm = jnp.full((ROW_BLK, COL_BLK), jnp.inf, dtype=jnp.float32)
    runc = jnp.zeros((ROW_BLK, COL_BLK), dtype=jnp.int32)
    for k in range(NUM_CODES // COL_BLK):
        cbt = cbt_ref[:, pl.ds(k * COL_BLK, COL_BLK)]
        mm2 = lax.dot_general(v2, cbt, (((1,), (0,)), ((), ())),
                              preferred_element_type=jnp.float32)
        cbsq = cbsq_ref[:, pl.ds(k * COL_BLK, COL_BLK)]
        d = (vsq + cbsq) + mm2
        lt = d < runm
        runm = jnp.where(lt, d, runm)
        runc = jnp.where(lt, jnp.int32(k), runc)
    rowmin = jnp.min(runm, axis=1)
    lane = lax.broadcasted_iota(jnp.int32, (ROW_BLK, COL_BLK), 1)
    gidx = runc * COL_BLK + lane
    cand = jnp.where(runm == rowmin[:, None], gidx, jnp.int32(2**30))
    idx_ref[0, 0, :] = jnp.min(cand, axis=1)

    @pl.when(pid == 0)
    def _init():
        vq_ref[...] = jnp.zeros((1, 1), jnp.float32)

    vq_ref[...] += jnp.sum(rowmin).reshape(1, 1)


def _nearest(vectors, codebook_t):
    n = vectors.shape[0]
    nblk = n // ROW_BLK
    idx3, vq_sum = pl.pallas_call(
        _argmin_body,
        grid=(nblk,),
        in_specs=[
            pl.BlockSpec((ROW_BLK, DIM), lambda i: (i, 0)),
            pl.BlockSpec((DIM, NUM_CODES), lambda i: (0, 0)),
        ],
        out_specs=[
            pl.BlockSpec((1, 1, ROW_BLK), lambda i: (i, 0, 0)),
            pl.BlockSpec((1, 1), lambda i: (0, 0)),
        ],
        out_shape=[
            jax.ShapeDtypeStruct((nblk, 1, ROW_BLK), jnp.int32),
            jax.ShapeDtypeStruct((1, 1), jnp.float32),
        ],
        scratch_shapes=[pltpu.VMEM((1, NUM_CODES), jnp.float32)],
    )(vectors, codebook_t)
    return idx3.reshape(n), vq_sum.reshape(())


_GCHUNK = 128
_PAD_D = 128


def _sc_gather_hist(n):
    per_w = n // _NW
    mesh = plsc.VectorSubcoreMesh(core_axis_name="c", subcore_axis_name="s")

    @functools.partial(
        pl.kernel,
        mesh=mesh,
        out_type=[
            jax.ShapeDtypeStruct((n, _PAD_D), jnp.float32),
            jax.ShapeDtypeStruct((_NW, NUM_CODES), jnp.float32),
        ],
        scratch_types=[
            pltpu.VMEM((per_w,), jnp.int32),
            pltpu.VMEM((per_w, _PAD_D), jnp.float32),
            pltpu.VMEM((NUM_CODES,), jnp.float32),
            pltpu.SemaphoreType.DMA,
        ],
        compiler_params=pltpu.CompilerParams(needs_layout_passes=False),
    )
    def body(idx_hbm, cb_hbm, rows_out, hist_out, idx_v, rows_v, hist_v, sem):
        wid = lax.axis_index("s") * _NC + lax.axis_index("c")
        base = wid * per_w
        pltpu.sync_copy(idx_hbm.at[pl.ds(base, per_w)], idx_v)
        copies = [
            pltpu.async_copy(
                cb_hbm.at[idx_v.at[pl.ds(k * _GCHUNK, _GCHUNK)]],
                rows_v.at[pl.ds(k * _GCHUNK, _GCHUNK)],
                sem,
            )
            for k in range(per_w // _GCHUNK)
        ]
        for c in copies:
            c.wait()
        pltpu.sync_copy(rows_v, rows_out.at[pl.ds(base, per_w)])

        def _zero(i, carry):
            hist_v[pl.ds(i * _L, _L)] = jnp.zeros((_L,), jnp.float32)
            return carry

        lax.fori_loop(0, NUM_CODES // _L, _zero, 0)
        ones = jnp.ones((_L,), jnp.float32)
        full = jnp.ones((_L,), jnp.bool_)

        def _acc(i, carry):
            iv = idx_v[pl.ds(i * _L, _L)]
            plsc.addupdate_scatter(hist_v, [iv], ones, mask=full)
            return carry

        lax.fori_loop(0, per_w // _L, _acc, 0)
        pltpu.sync_copy(hist_v, hist_out.at[wid])

    return body


def _stats_body(h_ref, perp_ref, act_ref):
    counts = jnp.sum(h_ref[...], axis=0, keepdims=True)
    total = jnp.sum(counts)
    probs = counts / total
    ent = jnp.sum(probs * jnp.log(probs + 1e-10))
    perp_ref[...] = jnp.exp(-ent).reshape(1, 1)
    act_ref[...] = jnp.sum((counts > 0).astype(jnp.float32)).reshape(1, 1)


def _stats(hist):
    perp, act = pl.pallas_call(
        _stats_body,
        out_shape=[
            jax.ShapeDtypeStruct((1, 1), jnp.float32),
            jax.ShapeDtypeStruct((1, 1), jnp.float32),
        ],
    )(hist)
    return perp.reshape(()), act.reshape(())


def kernel(z, codebook):
    B, C, H, W = z.shape
    n = B * H * W
    vectors = jnp.transpose(z, (0, 2, 3, 1)).reshape(n, C)
    indices, vq_sum = _nearest(vectors, codebook.T)

    cb_pad = jnp.pad(codebook, ((0, 0), (0, _PAD_D - C)))
    qrows, hist = _sc_gather_hist(n)(indices, cb_pad)

    quantized = jnp.transpose(qrows[:, :C].reshape(B, H, W, C), (0, 3, 1, 2))
    vq_loss = vq_sum / jnp.float32(n * C)
    perplexity, active = _stats(hist)
    return (quantized, indices, vq_loss, perplexity, active)
